# baseline (speedup 1.0000x reference)
import functools
import math

import numpy as np
import jax
import jax.numpy as jnp
from jax import lax
from jax.experimental import pallas as pl
from jax.experimental.pallas import tpu as pltpu
from jax.experimental.pallas import tpu_sc as plsc

DIM = 1024
SEQ = 2048
BATCH = 4
SCALE = math.sqrt(DIM)

NC, NS, L = 2, 16, 16
NW = NC * NS
ROWS = BATCH * SEQ
RPW = ROWS // NW
WPB = NW // BATCH
CH = 32
NCHUNK = RPW // CH
VPR = DIM // L


def _pos_enc() -> np.ndarray:
    pos = np.arange(SEQ, dtype=np.float64)[:, None]
    idx = np.arange(0, DIM, 2, dtype=np.float64)[None, :]
    angle = pos / (10000.0 ** (idx / DIM))
    pe = np.zeros((SEQ, DIM), dtype=np.float32)
    pe[:, 0::2] = np.sin(angle)
    pe[:, 1::2] = np.cos(angle)
    return pe


_PE = _pos_enc()


def _emb_body(x_hbm, tab_hbm, pe_hbm, out_hbm, idx_v, rows_v, pe_v, gsem, psem):
    wid = lax.axis_index("s") * NC + lax.axis_index("c")
    base = wid * RPW
    pbase = lax.rem(wid, WPB) * RPW

    pltpu.sync_copy(x_hbm.at[wid], idx_v)

    for j in range(NCHUNK):
        gcp = pltpu.async_copy(tab_hbm.at[idx_v.at[j]], rows_v, gsem)
        pcp = pltpu.async_copy(pe_hbm.at[pl.ds(pbase + j * CH, CH)], pe_v, psem)
        gcp.wait()
        pcp.wait()

        def _row(r, _):
            def _col(c, _):
                sl = pl.ds(c * L, L)
                rows_v[r, sl] = rows_v[r, sl] * SCALE + pe_v[r, sl]
                return 0
            lax.fori_loop(0, VPR, _col, 0)
            return 0
        lax.fori_loop(0, CH, _row, 0)

        pltpu.sync_copy(rows_v, out_hbm.at[pl.ds(base + j * CH, CH)])


@jax.jit
def kernel(x, emb_table):
    x3 = x.reshape(NW, NCHUNK, CH)
    mesh = plsc.VectorSubcoreMesh(core_axis_name="c", subcore_axis_name="s")
    run = functools.partial(
        pl.kernel,
        out_type=jax.ShapeDtypeStruct((ROWS, DIM), jnp.float32),
        mesh=mesh,
        scratch_types=[
            pltpu.VMEM((NCHUNK, CH), jnp.int32),
            pltpu.VMEM((CH, DIM), jnp.float32),
            pltpu.VMEM((CH, DIM), jnp.float32),
            pltpu.SemaphoreType.DMA,
            pltpu.SemaphoreType.DMA,
        ],
    )(_emb_body)
    out = run(x3, emb_table, _PE)
    return out.reshape(BATCH, SEQ, DIM)


# speedup vs baseline: 1.2128x; 1.2128x over previous
import functools
import math

import numpy as np
import jax
import jax.numpy as jnp
from jax import lax
from jax.experimental import pallas as pl
from jax.experimental.pallas import tpu as pltpu
from jax.experimental.pallas import tpu_sc as plsc

DIM = 1024
SEQ = 2048
BATCH = 4
SCALE = math.sqrt(DIM)

NC, NS, L = 2, 16, 16
NW = NC * NS
PPW = SEQ // NW
CHP = 8
CHR = CHP * BATCH
NCHUNK = PPW // CHP
VPR = DIM // L


def _pos_enc() -> np.ndarray:
    pos = np.arange(SEQ, dtype=np.float64)[:, None]
    idx = np.arange(0, DIM, 2, dtype=np.float64)[None, :]
    angle = pos / (10000.0 ** (idx / DIM))
    pe = np.zeros((SEQ, DIM), dtype=np.float32)
    pe[:, 0::2] = np.sin(angle)
    pe[:, 1::2] = np.cos(angle)
    return pe


_PE = _pos_enc()


def _emb_body(x_hbm, tab_hbm, pe_hbm, out_hbm,
              idx_v, buf, pe_v, gs0, gs1, ps0, ps1, os0, os1):
    gsem = (gs0, gs1)
    psem = (ps0, ps1)
    osem = (os0, os1)
    wid = lax.axis_index("s") * NC + lax.axis_index("c")
    p0 = wid * PPW

    pltpu.sync_copy(x_hbm.at[wid], idx_v)

    def start_chunk(j):
        slot = j % 2
        g = pltpu.async_copy(tab_hbm.at[idx_v.at[j]], buf.at[slot], gsem[slot])
        p = pltpu.async_copy(pe_hbm.at[pl.ds(p0 + j * CHP, CHP)],
                             pe_v.at[slot], psem[slot])
        return g, p

    def store_chunk(j):
        slot = j % 2
        cps = []
        for b in range(BATCH):
            cps.append(pltpu.async_copy(
                buf.at[slot, pl.ds(b * CHP, CHP)],
                out_hbm.at[pl.ds(b * SEQ + p0 + j * CHP, CHP)],
                osem[slot]))
        return cps

    def compute_chunk(j):
        slot = j % 2

        def _pos_body(p, _):
            def _col(c, _):
                sl = pl.ds(c * L, L)
                pv = pe_v[slot, p, sl]
                for b in range(BATCH):
                    r = b * CHP + p
                    buf[slot, r, sl] = buf[slot, r, sl] * SCALE + pv
                return 0
            lax.fori_loop(0, VPR, _col, 0)
            return 0
        lax.fori_loop(0, CHP, _pos_body, 0)

    pending_in = [None, None]
    pending_out = [None, None]
    pending_in[0] = start_chunk(0)
    for j in range(NCHUNK):
        slot = j % 2
        nxt = 1 - slot
        g, p = pending_in[slot]
        g.wait()
        p.wait()
        if j + 1 < NCHUNK:
            if pending_out[nxt] is not None:
                for cp in pending_out[nxt]:
                    cp.wait()
                pending_out[nxt] = None
            pending_in[nxt] = start_chunk(j + 1)
        compute_chunk(j)
        pending_out[slot] = store_chunk(j)
    for slot in range(2):
        if pending_out[slot] is not None:
            for cp in pending_out[slot]:
                cp.wait()


@jax.jit
def kernel(x, emb_table):
    x4 = x.reshape(BATCH, NW, NCHUNK, CHP).transpose(1, 2, 0, 3)
    x4 = x4.reshape(NW, NCHUNK, CHR)
    mesh = plsc.VectorSubcoreMesh(core_axis_name="c", subcore_axis_name="s")
    run = functools.partial(
        pl.kernel,
        out_type=jax.ShapeDtypeStruct((BATCH * SEQ, DIM), jnp.float32),
        mesh=mesh,
        scratch_types=[
            pltpu.VMEM((NCHUNK, CHR), jnp.int32),
            pltpu.VMEM((2, CHR, DIM), jnp.float32),
            pltpu.VMEM((2, CHP, DIM), jnp.float32),
            pltpu.SemaphoreType.DMA,
            pltpu.SemaphoreType.DMA,
            pltpu.SemaphoreType.DMA,
            pltpu.SemaphoreType.DMA,
            pltpu.SemaphoreType.DMA,
            pltpu.SemaphoreType.DMA,
        ],
    )(_emb_body)
    out = run(x4, emb_table, _PE)
    return out.reshape(BATCH, SEQ, DIM)


# speedup vs baseline: 2.8091x; 2.3162x over previous
import functools
import math

import numpy as np
import jax
import jax.numpy as jnp
from jax import lax
from jax.experimental import pallas as pl
from jax.experimental.pallas import tpu as pltpu
from jax.experimental.pallas import tpu_sc as plsc

DIM = 1024
SEQ = 2048
BATCH = 4
SCALE = math.sqrt(DIM)

NC, NS, L = 2, 16, 16
NW = NC * NS
PPW = SEQ // NW
CHP = 8
CHR = CHP * BATCH
NCHUNK = PPW // CHP
VPR = DIM // L


def _pos_enc() -> np.ndarray:
    pos = np.arange(SEQ, dtype=np.float64)[:, None]
    idx = np.arange(0, DIM, 2, dtype=np.float64)[None, :]
    angle = pos / (10000.0 ** (idx / DIM))
    pe = np.zeros((SEQ, DIM), dtype=np.float32)
    pe[:, 0::2] = np.sin(angle)
    pe[:, 1::2] = np.cos(angle)
    return pe


_PE = _pos_enc()


def _emb_body(x_hbm, tab_hbm, pe_hbm, out_hbm,
              idx_v, buf, pe_v, gs0, gs1, ps0, ps1, os0, os1):
    gsem = (gs0, gs1)
    psem = (ps0, ps1)
    osem = (os0, os1)
    wid = lax.axis_index("s") * NC + lax.axis_index("c")
    p0 = wid * PPW

    pltpu.sync_copy(x_hbm.at[wid], idx_v)

    def start_chunk(j):
        slot = j % 2
        g = pltpu.async_copy(tab_hbm.at[idx_v.at[j]], buf.at[slot], gsem[slot])
        p = pltpu.async_copy(pe_hbm.at[pl.ds(p0 + j * CHP, CHP)],
                             pe_v.at[slot], psem[slot])
        return g, p

    def store_chunk(j):
        slot = j % 2
        cps = []
        for b in range(BATCH):
            cps.append(pltpu.async_copy(
                buf.at[slot, pl.ds(b * CHP, CHP)],
                out_hbm.at[pl.ds(b * SEQ + p0 + j * CHP, CHP)],
                osem[slot]))
        return cps

    def compute_chunk(j):
        slot = j % 2

        @plsc.parallel_loop(0, CHP * VPR, unroll=4)
        def _body(i):
            p = lax.shift_right_logical(i, 6)
            c = lax.bitwise_and(i, VPR - 1)
            sl = pl.ds(c * L, L)
            pv = pe_v[slot, p, sl]
            for b in range(BATCH):
                r = b * CHP + p
                buf[slot, r, sl] = buf[slot, r, sl] * SCALE + pv

    pending_in = [None, None]
    pending_out = [None, None]
    pending_in[0] = start_chunk(0)
    for j in range(NCHUNK):
        slot = j % 2
        nxt = 1 - slot
        g, p = pending_in[slot]
        g.wait()
        p.wait()
        if j + 1 < NCHUNK:
            if pending_out[nxt] is not None:
                for cp in pending_out[nxt]:
                    cp.wait()
                pending_out[nxt] = None
            pending_in[nxt] = start_chunk(j + 1)
        compute_chunk(j)
        pending_out[slot] = store_chunk(j)
    for slot in range(2):
        if pending_out[slot] is not None:
            for cp in pending_out[slot]:
                cp.wait()


@jax.jit
def kernel(x, emb_table):
    x4 = x.reshape(BATCH, NW, NCHUNK, CHP).transpose(1, 2, 0, 3)
    x4 = x4.reshape(NW, NCHUNK, CHR)
    mesh = plsc.VectorSubcoreMesh(core_axis_name="c", subcore_axis_name="s")
    run = functools.partial(
        pl.kernel,
        out_type=jax.ShapeDtypeStruct((BATCH * SEQ, DIM), jnp.float32),
        mesh=mesh,
        scratch_types=[
            pltpu.VMEM((NCHUNK, CHR), jnp.int32),
            pltpu.VMEM((2, CHR, DIM), jnp.float32),
            pltpu.VMEM((2, CHP, DIM), jnp.float32),
            pltpu.SemaphoreType.DMA,
            pltpu.SemaphoreType.DMA,
            pltpu.SemaphoreType.DMA,
            pltpu.SemaphoreType.DMA,
            pltpu.SemaphoreType.DMA,
            pltpu.SemaphoreType.DMA,
        ],
    )(_emb_body)
    out = run(x4, emb_table, _PE)
    return out.reshape(BATCH, SEQ, DIM)
